# baseline (speedup 1.0000x reference)
import functools

import jax
import jax.numpy as jnp
from jax import lax
from jax.experimental import pallas as pl
from jax.experimental.pallas import tpu as pltpu
from jax.experimental.pallas import tpu_sc as plsc

B, L, V, D = 1024, 200, 100000, 128
NC, NS, LANES = 2, 16, 16
NW = NC * NS
SEQ_PER_W = B // NW
VECS_PER_ROW = D // LANES
C0 = 128
C1 = L - C0
NBUF = 3


def _sc_body(x_hbm, tok_hbm, pos_hbm, out_hbm,
             idx_all, pos_v, r0, r1, r2,
             gs0, gs1, gs2, os0, os1, os2, psem):
    rows = (r0, r1, r2)
    gsems = (gs0, gs1, gs2)
    osems = (os0, os1, os2)

    wid = lax.axis_index("s") * NC + lax.axis_index("c")
    seq0 = wid * SEQ_PER_W

    pos_cp = pltpu.async_copy(pos_hbm, pos_v, psem)
    pltpu.sync_copy(x_hbm.at[pl.ds(seq0, SEQ_PER_W)], idx_all)

    def start_gather(s):
        b = s % NBUF
        return (
            pltpu.async_copy(
                tok_hbm.at[idx_all.at[s, pl.ds(0, C0)]],
                rows[b].at[pl.ds(0, C0)], gsems[b]),
            pltpu.async_copy(
                tok_hbm.at[idx_all.at[s, pl.ds(C0, C1)]],
                rows[b].at[pl.ds(C0, C1)], gsems[b]),
        )

    gather_descs = {0: start_gather(0), 1: start_gather(1)}
    out_descs = {}

    pos_cp.wait()
    for s in range(SEQ_PER_W):
        b = s % NBUF
        for cp in gather_descs.pop(s):
            cp.wait()

        rows_b = rows[b]

        @plsc.parallel_loop(0, L, step=1, unroll=4)
        def add_body(r, rows_b=rows_b):
            for c in range(VECS_PER_ROW):
                sl = pl.ds(c * LANES, LANES)
                plsc.addupdate(rows_b.at[r, sl], pos_v[r, sl])

        out_descs[s] = pltpu.async_copy(rows_b, out_hbm.at[seq0 + s], osems[b])

        t = s + 2
        if t < SEQ_PER_W:
            if t >= NBUF:
                out_descs.pop(t - NBUF).wait()
            gather_descs[t] = start_gather(t)

    for s in sorted(out_descs):
        out_descs.pop(s).wait()


@jax.jit
def _run(x, token_table, pos_table):
    mesh = plsc.VectorSubcoreMesh(core_axis_name="c", subcore_axis_name="s")
    kfn = functools.partial(
        pl.kernel,
        mesh=mesh,
        out_type=jax.ShapeDtypeStruct((B, L, D), jnp.float32),
        scratch_types=[
            pltpu.VMEM((SEQ_PER_W, L), jnp.int32),
            pltpu.VMEM((L, D), jnp.float32),
            pltpu.VMEM((L, D), jnp.float32),
            pltpu.VMEM((L, D), jnp.float32),
            pltpu.VMEM((L, D), jnp.float32),
            pltpu.SemaphoreType.DMA,
            pltpu.SemaphoreType.DMA,
            pltpu.SemaphoreType.DMA,
            pltpu.SemaphoreType.DMA,
            pltpu.SemaphoreType.DMA,
            pltpu.SemaphoreType.DMA,
            pltpu.SemaphoreType.DMA,
        ],
    )(_sc_body)
    return kfn(x, token_table, pos_table)


def kernel(x, token_table, pos_table):
    return _run(x.astype(jnp.int32), token_table, pos_table)


# speedup vs baseline: 1.0100x; 1.0100x over previous
import functools

import jax
import jax.numpy as jnp
from jax import lax
from jax.experimental import pallas as pl
from jax.experimental.pallas import tpu as pltpu
from jax.experimental.pallas import tpu_sc as plsc

B, L, V, D = 1024, 200, 100000, 128
NC, NS, LANES = 2, 16, 16
NW = NC * NS
SEQ_PER_W = B // NW
VECS_PER_ROW = D // LANES
G = 4
PH = 5
RPC = L // PH
GROUPS = SEQ_PER_W // G
STEPS = PH * GROUPS
NBUF = 4


def _sc_body(x_hbm, tok_hbm, pos_hbm, out_hbm,
             idx_all, pos_v, b0, b1, b2, b3,
             gs0, gs1, gs2, gs3, os0, os1, os2, os3, psem):
    bufs = (b0, b1, b2, b3)
    gsems = (gs0, gs1, gs2, gs3)
    osems = (os0, os1, os2, os3)

    wid = lax.axis_index("s") * NC + lax.axis_index("c")
    seq0 = wid * SEQ_PER_W

    pos_cp = pltpu.async_copy(pos_hbm, pos_v, psem)
    pltpu.sync_copy(x_hbm.at[pl.ds(seq0 * L, SEQ_PER_W * L)], idx_all)

    def start_gathers(k):
        c, g = k // GROUPS, k % GROUPS
        bset = bufs[k % NBUF]
        sem = gsems[k % NBUF]
        return tuple(
            pltpu.async_copy(
                tok_hbm.at[idx_all.at[pl.ds((g * G + j) * L + c * RPC, RPC)]],
                bset.at[j], sem)
            for j in range(G)
        )

    def start_outs(k):
        c, g = k // GROUPS, k % GROUPS
        bset = bufs[k % NBUF]
        sem = osems[k % NBUF]
        return tuple(
            pltpu.async_copy(
                bset.at[j],
                out_hbm.at[pl.ds((seq0 + g * G + j) * L + c * RPC, RPC)],
                sem)
            for j in range(G)
        )

    gather_descs = {0: start_gathers(0), 1: start_gathers(1)}
    out_descs = {}

    pos_cp.wait()
    for k in range(STEPS):
        c = k // GROUPS
        bset = bufs[k % NBUF]
        for cp in gather_descs.pop(k):
            cp.wait()

        @plsc.parallel_loop(0, RPC, step=1, unroll=1)
        def add_body(r, bset=bset, c=c):
            pvs = [
                pos_v[c * RPC + r, pl.ds(ci * LANES, LANES)]
                for ci in range(VECS_PER_ROW)
            ]
            for j in range(G):
                for ci in range(VECS_PER_ROW):
                    plsc.addupdate(
                        bset.at[j, r, pl.ds(ci * LANES, LANES)], pvs[ci])

        out_descs[k] = start_outs(k)

        t = k + 2
        if t < STEPS:
            if t >= NBUF:
                for cp in out_descs.pop(t - NBUF):
                    cp.wait()
            gather_descs[t] = start_gathers(t)

    for k in sorted(out_descs):
        for cp in out_descs.pop(k):
            cp.wait()


@jax.jit
def _run(x, token_table, pos_table):
    mesh = plsc.VectorSubcoreMesh(core_axis_name="c", subcore_axis_name="s")
    kfn = functools.partial(
        pl.kernel,
        mesh=mesh,
        out_type=jax.ShapeDtypeStruct((B * L, D), jnp.float32),
        scratch_types=[
            pltpu.VMEM((SEQ_PER_W * L,), jnp.int32),
            pltpu.VMEM((L, D), jnp.float32),
            pltpu.VMEM((G, RPC, D), jnp.float32),
            pltpu.VMEM((G, RPC, D), jnp.float32),
            pltpu.VMEM((G, RPC, D), jnp.float32),
            pltpu.VMEM((G, RPC, D), jnp.float32),
            pltpu.SemaphoreType.DMA,
            pltpu.SemaphoreType.DMA,
            pltpu.SemaphoreType.DMA,
            pltpu.SemaphoreType.DMA,
            pltpu.SemaphoreType.DMA,
            pltpu.SemaphoreType.DMA,
            pltpu.SemaphoreType.DMA,
            pltpu.SemaphoreType.DMA,
            pltpu.SemaphoreType.DMA,
        ],
    )(_sc_body)
    return kfn(x, token_table, pos_table)


def kernel(x, token_table, pos_table):
    out = _run(x.astype(jnp.int32).reshape(B * L), token_table, pos_table)
    return out.reshape(B, L, D)
